# baseline (speedup 1.0000x reference)
import functools

import jax
import jax.numpy as jnp
from jax import lax
from jax.experimental import pallas as pl
from jax.experimental.pallas import tpu as pltpu
from jax.experimental.pallas import tpu_sc as plsc

V = 100000
D = 64
K = 8
DK = D * K
B = 16384

NC = 2
NS = 16
NW = NC * NS
L = 16

BPW = B // NW
CHUNK = 64
NCHUNK = BPW // CHUNK
GROUPS = CHUNK // L
NBUF = 2
NST = 3


def _splat(val, dtype=jnp.int32):
    return jnp.full((L,), val, dtype=dtype)


def _sense_kernel(word_hbm, ctxh_hbm, ctxo_hbm, tgth_hbm, tgto_hbm,
                  wg_hbm, ws_hbm, out_hbm, *scratch):
    ctxo_v, tgto_v = scratch[0:2]
    word_v = scratch[2:5]
    ctxh_v = scratch[5:8]
    tgth_v = scratch[8:11]
    sense_v = scratch[11:13]
    ctxr_v = scratch[13:15]
    tgtr_v = scratch[15:17]
    out_v = scratch[17]
    sems = scratch[18:20]
    isems = scratch[20:23]

    wid = lax.axis_index("s") * NC + lax.axis_index("c")
    base0 = wid * BPW

    pltpu.sync_copy(ctxo_hbm.at[pl.ds(base0, BPW)], ctxo_v)
    pltpu.sync_copy(tgto_hbm.at[pl.ds(base0, BPW)], tgto_v)

    def stage(step):
        ib = step % NST
        base = base0 + step * CHUNK
        s1 = pltpu.async_copy(word_hbm.at[pl.ds(base, CHUNK)],
                              word_v[ib], isems[ib])
        s2 = pltpu.async_copy(ctxh_hbm.at[pl.ds(base, CHUNK)],
                              ctxh_v[ib], isems[ib])
        s3 = pltpu.async_copy(tgth_hbm.at[pl.ds(base, CHUNK)],
                              tgth_v[ib], isems[ib])
        return (s1, s2, s3)

    def gather(step, stcopies):
        ib = step % NST
        b = step % NBUF
        for c in stcopies:
            c.wait()
        c1 = pltpu.async_copy(ws_hbm.at[word_v[ib]], sense_v[b], sems[b])
        c2 = pltpu.async_copy(wg_hbm.at[ctxh_v[ib]], ctxr_v[b], sems[b])
        c3 = pltpu.async_copy(wg_hbm.at[tgth_v[ib]], tgtr_v[b], sems[b])
        return (c1, c2, c3)

    iota = lax.iota(jnp.int32, L)
    zv = _splat(0)
    zf = _splat(0.0, jnp.float32)

    def compute_chunk(step, b):
        def group_body(g, _):
            tok = g * L + iota
            s0 = step * CHUNK + g * L
            ctxoff = ctxo_v[pl.ds(s0, L)]
            tgtoff = tgto_v[pl.ds(s0, L)]

            def score_body(d, carry):
                accs = carry[0:K]
                scol, ccol = carry[K], carry[K + 1]
                ctxv = plsc.load_gather(ctxr_v[b], [tok, ccol])
                new = []
                for k in range(K):
                    sv = plsc.load_gather(sense_v[b], [tok, scol + k])
                    new.append(accs[k] + ctxv * sv)
                return tuple(new) + (scol + K, ccol + 1)

            init = (zf,) * K + (zv, ctxoff)
            res = lax.fori_loop(0, D, score_body, init, unroll=8)
            accs = res[0:K]

            best = accs[0]
            bidx = zv
            for k in range(1, K):
                m = accs[k] > best
                best = jnp.where(m, accs[k], best)
                bidx = jnp.where(m, _splat(k), bidx)

            def dot_body(d, carry):
                acc, dcol, tcol = carry
                chosen = plsc.load_gather(sense_v[b], [tok, dcol])
                tv = plsc.load_gather(tgtr_v[b], [tok, tcol])
                return (acc + chosen * tv, dcol + K, tcol + 1)

            dinit = (zf, bidx, tgtoff)
            dot, _, _ = lax.fori_loop(0, D, dot_body, dinit, unroll=8)
            res = 1.0 / (1.0 + jnp.exp(-dot))
            out_v[pl.ds(s0, L)] = res
            return 0

        lax.fori_loop(0, GROUPS, group_body, 0)

    st = [stage(0), stage(1), stage(2)]
    gcop = [gather(0, st[0]), gather(1, st[1])]
    for step in range(NCHUNK):
        b = step % NBUF
        for c in gcop[b]:
            c.wait()
        compute_chunk(step, b)
        if step + NST < NCHUNK:
            st[(step + NST) % NST] = stage(step + NST)
        if step + NBUF < NCHUNK:
            gcop[b] = gather(step + NBUF, st[(step + NBUF) % NST])
    pltpu.sync_copy(out_v, out_hbm.at[pl.ds(base0, BPW)])


@jax.jit
def _run(word, ctx_hi, ctx_off, tgt_hi, tgt_off, wg2, ws2):
    mesh = plsc.VectorSubcoreMesh(core_axis_name="c", subcore_axis_name="s")
    idx_t = pltpu.VMEM((BPW,), jnp.int32)
    f = functools.partial(
        pl.kernel,
        mesh=mesh,
        compiler_params=pltpu.CompilerParams(needs_layout_passes=False),
        out_type=jax.ShapeDtypeStruct((B,), jnp.float32),
        scratch_types=[idx_t] * 2 + [pltpu.VMEM((CHUNK,), jnp.int32)] * 9 + [
            pltpu.VMEM((CHUNK, DK), jnp.float32),
            pltpu.VMEM((CHUNK, DK), jnp.float32),
            pltpu.VMEM((CHUNK, 2 * D), jnp.float32),
            pltpu.VMEM((CHUNK, 2 * D), jnp.float32),
            pltpu.VMEM((CHUNK, 2 * D), jnp.float32),
            pltpu.VMEM((CHUNK, 2 * D), jnp.float32),
            pltpu.VMEM((BPW,), jnp.float32),
            pltpu.SemaphoreType.DMA,
            pltpu.SemaphoreType.DMA,
            pltpu.SemaphoreType.DMA,
            pltpu.SemaphoreType.DMA,
            pltpu.SemaphoreType.DMA,
        ],
    )(_sense_kernel)
    return f(word, ctx_hi, ctx_off, tgt_hi, tgt_off, wg2, ws2)


def kernel(x, W_g, W_s):
    word = x[0].astype(jnp.int32)
    ctx = x[1].astype(jnp.int32)
    tgt = x[2].astype(jnp.int32)
    ctx_hi = ctx >> 1
    ctx_off = (ctx & 1) * D
    tgt_hi = tgt >> 1
    tgt_off = (tgt & 1) * D
    wg2 = W_g.reshape(V // 2, 2 * D)
    ws2 = W_s.reshape(V, D * K)
    return _run(word, ctx_hi, ctx_off, tgt_hi, tgt_off, wg2, ws2)


# speedup vs baseline: 1.0047x; 1.0047x over previous
import functools

import jax
import jax.numpy as jnp
from jax import lax
from jax.experimental import pallas as pl
from jax.experimental.pallas import tpu as pltpu
from jax.experimental.pallas import tpu_sc as plsc

V = 100000
D = 64
K = 8
DK = D * K
B = 16384

NC = 2
NS = 16
NW = NC * NS
L = 16

BPW = B // NW
CHUNK = 64
NCHUNK = BPW // CHUNK
GROUPS = CHUNK // L
NBUF = 2
NST = 3


def _splat(val, dtype=jnp.int32):
    return jnp.full((L,), val, dtype=dtype)


def _sense_kernel(word_hbm, ctxh_hbm, ctxo_hbm, tgth_hbm, tgto_hbm,
                  wg_hbm, ws_hbm, out_hbm, *scratch):
    ctxo_v, tgto_v = scratch[0:2]
    word_v = scratch[2:5]
    ctxh_v = scratch[5:8]
    tgth_v = scratch[8:11]
    sense_v = scratch[11:13]
    ctxr_v = scratch[13:15]
    tgtr_v = scratch[15:17]
    out_v = scratch[17]
    sems = scratch[18:20]
    isems = scratch[20:23]

    wid = lax.axis_index("s") * NC + lax.axis_index("c")
    base0 = wid * BPW

    pltpu.sync_copy(ctxo_hbm.at[pl.ds(base0, BPW)], ctxo_v)
    pltpu.sync_copy(tgto_hbm.at[pl.ds(base0, BPW)], tgto_v)

    def stage(step):
        ib = step % NST
        base = base0 + step * CHUNK
        s1 = pltpu.async_copy(word_hbm.at[pl.ds(base, CHUNK)],
                              word_v[ib], isems[ib])
        s2 = pltpu.async_copy(ctxh_hbm.at[pl.ds(base, CHUNK)],
                              ctxh_v[ib], isems[ib])
        s3 = pltpu.async_copy(tgth_hbm.at[pl.ds(base, CHUNK)],
                              tgth_v[ib], isems[ib])
        return (s1, s2, s3)

    def gather(step, stcopies):
        ib = step % NST
        b = step % NBUF
        for c in stcopies:
            c.wait()
        c1 = pltpu.async_copy(ws_hbm.at[word_v[ib]], sense_v[b], sems[b])
        c2 = pltpu.async_copy(wg_hbm.at[ctxh_v[ib]], ctxr_v[b], sems[b])
        c3 = pltpu.async_copy(wg_hbm.at[tgth_v[ib]], tgtr_v[b], sems[b])
        return (c1, c2, c3)

    iota = lax.iota(jnp.int32, L)
    zv = _splat(0)
    zf = _splat(0.0, jnp.float32)

    def compute_chunk(step, b):
        def group_body(g, _):
            tok = g * L + iota
            s0 = step * CHUNK + g * L
            ctxoff = ctxo_v[pl.ds(s0, L)]
            tgtoff = tgto_v[pl.ds(s0, L)]

            def score_body(d, carry):
                accs = carry[0:K]
                scol, ccol = carry[K], carry[K + 1]
                ctxv = plsc.load_gather(ctxr_v[b], [tok, ccol])
                new = []
                for k in range(K):
                    sv = plsc.load_gather(sense_v[b], [tok, scol + k])
                    new.append(accs[k] + ctxv * sv)
                return tuple(new) + (scol + K, ccol + 1)

            init = (zf,) * K + (zv, ctxoff)
            res = lax.fori_loop(0, D, score_body, init, unroll=4)
            accs = res[0:K]

            best = accs[0]
            bidx = zv
            for k in range(1, K):
                m = accs[k] > best
                best = jnp.where(m, accs[k], best)
                bidx = jnp.where(m, _splat(k), bidx)

            def dot_body(d, carry):
                acc, dcol, tcol = carry
                chosen = plsc.load_gather(sense_v[b], [tok, dcol])
                tv = plsc.load_gather(tgtr_v[b], [tok, tcol])
                return (acc + chosen * tv, dcol + K, tcol + 1)

            dinit = (zf, bidx, tgtoff)
            dot, _, _ = lax.fori_loop(0, D, dot_body, dinit, unroll=8)
            res = 1.0 / (1.0 + jnp.exp(-dot))
            out_v[pl.ds(s0, L)] = res
            return 0

        lax.fori_loop(0, GROUPS, group_body, 0)

    st = [stage(0), stage(1), stage(2)]
    gcop = [gather(0, st[0]), gather(1, st[1])]
    for step in range(NCHUNK):
        b = step % NBUF
        for c in gcop[b]:
            c.wait()
        compute_chunk(step, b)
        if step + NST < NCHUNK:
            st[(step + NST) % NST] = stage(step + NST)
        if step + NBUF < NCHUNK:
            gcop[b] = gather(step + NBUF, st[(step + NBUF) % NST])
    pltpu.sync_copy(out_v, out_hbm.at[pl.ds(base0, BPW)])


@jax.jit
def _run(word, ctx_hi, ctx_off, tgt_hi, tgt_off, wg2, ws2):
    mesh = plsc.VectorSubcoreMesh(core_axis_name="c", subcore_axis_name="s")
    idx_t = pltpu.VMEM((BPW,), jnp.int32)
    f = functools.partial(
        pl.kernel,
        mesh=mesh,
        compiler_params=pltpu.CompilerParams(needs_layout_passes=False),
        out_type=jax.ShapeDtypeStruct((B,), jnp.float32),
        scratch_types=[idx_t] * 2 + [pltpu.VMEM((CHUNK,), jnp.int32)] * 9 + [
            pltpu.VMEM((CHUNK, DK), jnp.float32),
            pltpu.VMEM((CHUNK, DK), jnp.float32),
            pltpu.VMEM((CHUNK, 2 * D), jnp.float32),
            pltpu.VMEM((CHUNK, 2 * D), jnp.float32),
            pltpu.VMEM((CHUNK, 2 * D), jnp.float32),
            pltpu.VMEM((CHUNK, 2 * D), jnp.float32),
            pltpu.VMEM((BPW,), jnp.float32),
            pltpu.SemaphoreType.DMA,
            pltpu.SemaphoreType.DMA,
            pltpu.SemaphoreType.DMA,
            pltpu.SemaphoreType.DMA,
            pltpu.SemaphoreType.DMA,
        ],
    )(_sense_kernel)
    return f(word, ctx_hi, ctx_off, tgt_hi, tgt_off, wg2, ws2)


def kernel(x, W_g, W_s):
    word = x[0].astype(jnp.int32)
    ctx = x[1].astype(jnp.int32)
    tgt = x[2].astype(jnp.int32)
    ctx_hi = ctx >> 1
    ctx_off = (ctx & 1) * D
    tgt_hi = tgt >> 1
    tgt_off = (tgt & 1) * D
    wg2 = W_g.reshape(V // 2, 2 * D)
    ws2 = W_s.reshape(V, D * K)
    return _run(word, ctx_hi, ctx_off, tgt_hi, tgt_off, wg2, ws2)
